# Initial kernel scaffold; baseline (speedup 1.0000x reference)
import jax
import jax.numpy as jnp
from jax.experimental import pallas as pl


def kernel(input_ids, position_ids, types_ids, tok_table, pos_table, typ_table, ln_gamma, ln_beta):
    raise NotImplementedError("write your pallas kernel here")


# speedup vs baseline: 22.3083x; 22.3083x over previous
import functools

import jax
import jax.numpy as jnp
from jax import lax
from jax.experimental import pallas as pl
from jax.experimental.pallas import tpu as pltpu
from jax.experimental.pallas import tpu_sc as plsc

B, L, D = 4096, 200, 128
V_TOK, V_POS, V_TYP = 13, 200, 2
V_FUSED = V_TOK * V_POS * V_TYP
N = B * L
EPS = 1e-5

NC, NS = 2, 16
NW = NC * NS
PER_W = N // NW
CHUNK = 512
GPC = CHUNK // 128


def _fused_table_body(tok_ref, pos_ref, typ_ref, gamma_ref, beta_ref, out_ref):
    jt = lax.broadcasted_iota(jnp.int32, (V_FUSED, V_TOK), 0)
    ct = lax.broadcasted_iota(jnp.int32, (V_FUSED, V_TOK), 1)
    oh_tok = (jt // (V_POS * V_TYP) == ct).astype(jnp.float32)
    jp = lax.broadcasted_iota(jnp.int32, (V_FUSED, V_POS), 0)
    cp = lax.broadcasted_iota(jnp.int32, (V_FUSED, V_POS), 1)
    oh_pos = ((jp // V_TYP) % V_POS == cp).astype(jnp.float32)
    e = jnp.dot(oh_tok, tok_ref[:], preferred_element_type=jnp.float32,
                precision=lax.Precision.HIGHEST)
    e = e + jnp.dot(oh_pos, pos_ref[:], preferred_element_type=jnp.float32,
                    precision=lax.Precision.HIGHEST)
    jd = lax.broadcasted_iota(jnp.int32, (V_FUSED, D), 0)
    e = e + jnp.where(jd % V_TYP == 0, typ_ref[0:1, :], typ_ref[1:2, :])
    mean = jnp.mean(e, axis=1, keepdims=True)
    cen = e - mean
    var = jnp.mean(cen * cen, axis=1, keepdims=True)
    normed = cen * lax.rsqrt(var + EPS)
    out_ref[:] = normed * gamma_ref[:] + beta_ref[:]


def _idx_body(tok_ref, pos_ref, typ_ref, out_ref):
    out_ref[:] = (tok_ref[:] * (V_POS * V_TYP) + pos_ref[:] * V_TYP
                  + typ_ref[:])


def _gather_body(idx_hbm, fused_hbm, out_hbm, idx_v, rows_v, sem):
    wid = lax.axis_index("s") * NC + lax.axis_index("c")
    base = wid * PER_W

    def chunk(i, carry):
        off = base + i * CHUNK
        pltpu.sync_copy(idx_hbm.at[pl.ds(off, CHUNK)], idx_v)
        handles = [
            pltpu.async_copy(
                fused_hbm.at[idx_v.at[pl.ds(j * 128, 128)]],
                rows_v.at[pl.ds(j * 128, 128)],
                sem,
            )
            for j in range(GPC)
        ]
        for h in handles:
            h.wait()
        pltpu.sync_copy(rows_v, out_hbm.at[pl.ds(off, CHUNK)])
        return carry

    lax.fori_loop(0, PER_W // CHUNK, chunk, 0)


@functools.cache
def _gather_call():
    return pl.kernel(
        _gather_body,
        out_type=jax.ShapeDtypeStruct((N, D), jnp.float32),
        mesh=plsc.VectorSubcoreMesh(core_axis_name="c", subcore_axis_name="s",
                                    num_cores=NC, num_subcores=NS),
        scratch_types=[
            pltpu.VMEM((CHUNK,), jnp.int32),
            pltpu.VMEM((CHUNK, D), jnp.float32),
            pltpu.SemaphoreType.DMA,
        ],
    )


def kernel(input_ids, position_ids, types_ids, tok_table, pos_table,
           typ_table, ln_gamma, ln_beta):
    fused = pl.pallas_call(
        _fused_table_body,
        out_shape=jax.ShapeDtypeStruct((V_FUSED, D), jnp.float32),
    )(tok_table, pos_table, typ_table,
      ln_gamma.reshape(1, D), ln_beta.reshape(1, D))

    blk = 512
    idx2d = pl.pallas_call(
        _idx_body,
        grid=(B // blk,),
        in_specs=[pl.BlockSpec((blk, L), lambda i: (i, 0))] * 3,
        out_specs=pl.BlockSpec((blk, L), lambda i: (i, 0)),
        out_shape=jax.ShapeDtypeStruct((B, L), jnp.int32),
    )(input_ids.astype(jnp.int32), position_ids.astype(jnp.int32),
      types_ids.astype(jnp.int32))

    out = _gather_call()(idx2d.reshape(N), fused)
    return out.reshape(B, L, D)


# speedup vs baseline: 23.5791x; 1.0570x over previous
import functools

import jax
import jax.numpy as jnp
from jax import lax
from jax.experimental import pallas as pl
from jax.experimental.pallas import tpu as pltpu
from jax.experimental.pallas import tpu_sc as plsc

B, L, D = 4096, 200, 128
V_TOK, V_POS, V_TYP = 13, 200, 2
V_FUSED = V_TOK * V_POS * V_TYP
N = B * L
EPS = 1e-5

NC, NS = 2, 16
NW = NC * NS
PER_W = N // NW
CHUNK = 128
NBUF = 4
NCHUNK = PER_W // CHUNK
NGRP = NCHUNK // NBUF


def _fused_table_body(tok_ref, pos_ref, typ_ref, gamma_ref, beta_ref, out_ref):
    jt = lax.broadcasted_iota(jnp.int32, (V_FUSED, V_TOK), 0)
    ct = lax.broadcasted_iota(jnp.int32, (V_FUSED, V_TOK), 1)
    oh_tok = (jt // (V_POS * V_TYP) == ct).astype(jnp.float32)
    jp = lax.broadcasted_iota(jnp.int32, (V_FUSED, V_POS), 0)
    cp = lax.broadcasted_iota(jnp.int32, (V_FUSED, V_POS), 1)
    oh_pos = ((jp // V_TYP) % V_POS == cp).astype(jnp.float32)
    e = jnp.dot(oh_tok, tok_ref[:], preferred_element_type=jnp.float32,
                precision=lax.Precision.HIGHEST)
    e = e + jnp.dot(oh_pos, pos_ref[:], preferred_element_type=jnp.float32,
                    precision=lax.Precision.HIGHEST)
    jd = lax.broadcasted_iota(jnp.int32, (V_FUSED, D), 0)
    e = e + jnp.where(jd % V_TYP == 0, typ_ref[0:1, :], typ_ref[1:2, :])
    mean = jnp.mean(e, axis=1, keepdims=True)
    cen = e - mean
    var = jnp.mean(cen * cen, axis=1, keepdims=True)
    normed = cen * lax.rsqrt(var + EPS)
    out_ref[:] = normed * gamma_ref[:] + beta_ref[:]


def _idx_body(tok_ref, pos_ref, typ_ref, out_ref):
    out_ref[:] = (tok_ref[:] * (V_POS * V_TYP) + pos_ref[:] * V_TYP
                  + typ_ref[:])


def _gather_body(idx_hbm, fused_hbm, out_hbm, idx_all, rows,
                 g0, g1, g2, g3, s0, s1, s2, s3):
    gsem = [g0, g1, g2, g3]
    ssem = [s0, s1, s2, s3]
    wid = lax.axis_index("s") * NC + lax.axis_index("c")
    base = wid * PER_W
    pltpu.sync_copy(idx_hbm.at[pl.ds(base, PER_W)], idx_all)

    def do_group(g, first):
        handles = []
        for b in range(NBUF):
            c = g * NBUF + b
            if not first:
                pltpu.make_async_copy(
                    rows.at[b], out_hbm.at[pl.ds(base + c * CHUNK, CHUNK)],
                    ssem[b]).wait()
            handles.append(pltpu.async_copy(
                fused_hbm.at[idx_all.at[pl.ds(c * CHUNK, CHUNK)]],
                rows.at[b], gsem[b]))
        for b in range(NBUF):
            c = g * NBUF + b
            handles[b].wait()
            pltpu.async_copy(rows.at[b],
                             out_hbm.at[pl.ds(base + c * CHUNK, CHUNK)],
                             ssem[b])

    do_group(0, True)

    def body(g, carry):
        do_group(g, False)
        return carry

    lax.fori_loop(1, NGRP, body, 0)
    for b in range(NBUF):
        pltpu.make_async_copy(
            rows.at[b], out_hbm.at[pl.ds(base + b * CHUNK, CHUNK)],
            ssem[b]).wait()


@functools.cache
def _gather_call():
    return pl.kernel(
        _gather_body,
        out_type=jax.ShapeDtypeStruct((N, D), jnp.float32),
        mesh=plsc.VectorSubcoreMesh(core_axis_name="c", subcore_axis_name="s",
                                    num_cores=NC, num_subcores=NS),
        scratch_types=[
            pltpu.VMEM((PER_W,), jnp.int32),
            pltpu.VMEM((NBUF, CHUNK, D), jnp.float32),
        ] + [pltpu.SemaphoreType.DMA] * (2 * NBUF),
    )


def kernel(input_ids, position_ids, types_ids, tok_table, pos_table,
           typ_table, ln_gamma, ln_beta):
    fused = pl.pallas_call(
        _fused_table_body,
        out_shape=jax.ShapeDtypeStruct((V_FUSED, D), jnp.float32),
    )(tok_table, pos_table, typ_table,
      ln_gamma.reshape(1, D), ln_beta.reshape(1, D))

    blk = 512
    idx2d = pl.pallas_call(
        _idx_body,
        grid=(B // blk,),
        in_specs=[pl.BlockSpec((blk, L), lambda i: (i, 0))] * 3,
        out_specs=pl.BlockSpec((blk, L), lambda i: (i, 0)),
        out_shape=jax.ShapeDtypeStruct((B, L), jnp.int32),
    )(input_ids.astype(jnp.int32), position_ids.astype(jnp.int32),
      types_ids.astype(jnp.int32))

    out = _gather_call()(idx2d.reshape(N), fused)
    return out.reshape(B, L, D)


# speedup vs baseline: 30.1519x; 1.2788x over previous
import functools

import jax
import jax.numpy as jnp
from jax import lax
from jax.experimental import pallas as pl
from jax.experimental.pallas import tpu as pltpu
from jax.experimental.pallas import tpu_sc as plsc

B, L, D = 4096, 200, 128
V_TOK, V_POS, V_TYP = 13, 200, 2
V_FUSED = V_TOK * V_POS * V_TYP
N = B * L
EPS = 1e-5

NC, NS = 2, 16
NW = NC * NS
PER_W = N // NW
CHUNK = 128
NBUF = 2
NCHUNK = PER_W // CHUNK
NGRP = NCHUNK // NBUF


def _fused_table_body(tok_ref, pos_ref, typ_ref, gamma_ref, beta_ref, out_ref):
    jt = lax.broadcasted_iota(jnp.int32, (V_FUSED, V_TOK), 0)
    ct = lax.broadcasted_iota(jnp.int32, (V_FUSED, V_TOK), 1)
    oh_tok = (jt // (V_POS * V_TYP) == ct).astype(jnp.float32)
    jp = lax.broadcasted_iota(jnp.int32, (V_FUSED, V_POS), 0)
    cp = lax.broadcasted_iota(jnp.int32, (V_FUSED, V_POS), 1)
    oh_pos = ((jp // V_TYP) % V_POS == cp).astype(jnp.float32)
    e = jnp.dot(oh_tok, tok_ref[:], preferred_element_type=jnp.float32,
                precision=lax.Precision.HIGHEST)
    e = e + jnp.dot(oh_pos, pos_ref[:], preferred_element_type=jnp.float32,
                    precision=lax.Precision.HIGHEST)
    jd = lax.broadcasted_iota(jnp.int32, (V_FUSED, D), 0)
    e = e + jnp.where(jd % V_TYP == 0, typ_ref[0:1, :], typ_ref[1:2, :])
    mean = jnp.mean(e, axis=1, keepdims=True)
    cen = e - mean
    var = jnp.mean(cen * cen, axis=1, keepdims=True)
    normed = cen * lax.rsqrt(var + EPS)
    out_ref[:] = normed * gamma_ref[:] + beta_ref[:]


def _idx_body(tok_ref, pos_ref, typ_ref, out_ref):
    out_ref[:] = (tok_ref[:] * (V_POS * V_TYP) + pos_ref[:] * V_TYP
                  + typ_ref[:])


def _gather_body(idx_hbm, fused_hbm, out_hbm, idx_all, rows, fused_sh,
                 g0, g1, s0, s1):
    gsem = [g0, g1]
    ssem = [s0, s1]
    sid = lax.axis_index("s")
    wid = sid * NC + lax.axis_index("c")
    base = wid * PER_W

    @pl.when(sid == 0)
    def _():
        pltpu.sync_copy(fused_hbm, fused_sh)

    pltpu.sync_copy(idx_hbm.at[pl.ds(base, PER_W)], idx_all)
    plsc.subcore_barrier()

    def do_group(g, first):
        handles = []
        for b in range(NBUF):
            c = g * NBUF + b
            if not first:
                pltpu.make_async_copy(
                    rows.at[b], out_hbm.at[pl.ds(base + c * CHUNK, CHUNK)],
                    ssem[b]).wait()
            handles.append(pltpu.async_copy(
                fused_sh.at[idx_all.at[pl.ds(c * CHUNK, CHUNK)]],
                rows.at[b], gsem[b]))
        for b in range(NBUF):
            c = g * NBUF + b
            handles[b].wait()
            pltpu.async_copy(rows.at[b],
                             out_hbm.at[pl.ds(base + c * CHUNK, CHUNK)],
                             ssem[b])

    do_group(0, True)

    def body(g, carry):
        do_group(g, False)
        return carry

    lax.fori_loop(1, NGRP, body, 0)
    for b in range(NBUF):
        pltpu.make_async_copy(
            rows.at[b], out_hbm.at[pl.ds(base + b * CHUNK, CHUNK)],
            ssem[b]).wait()


@functools.cache
def _gather_call():
    return pl.kernel(
        _gather_body,
        out_type=jax.ShapeDtypeStruct((N, D), jnp.float32),
        mesh=plsc.VectorSubcoreMesh(core_axis_name="c", subcore_axis_name="s",
                                    num_cores=NC, num_subcores=NS),
        scratch_types=[
            pltpu.VMEM((PER_W,), jnp.int32),
            pltpu.VMEM((NBUF, CHUNK, D), jnp.float32),
            pltpu.VMEM_SHARED((V_FUSED, D), jnp.float32),
        ] + [pltpu.SemaphoreType.DMA] * (2 * NBUF),
    )


def kernel(input_ids, position_ids, types_ids, tok_table, pos_table,
           typ_table, ln_gamma, ln_beta):
    fused = pl.pallas_call(
        _fused_table_body,
        out_shape=jax.ShapeDtypeStruct((V_FUSED, D), jnp.float32),
    )(tok_table, pos_table, typ_table,
      ln_gamma.reshape(1, D), ln_beta.reshape(1, D))

    blk = 512
    idx2d = pl.pallas_call(
        _idx_body,
        grid=(B // blk,),
        in_specs=[pl.BlockSpec((blk, L), lambda i: (i, 0))] * 3,
        out_specs=pl.BlockSpec((blk, L), lambda i: (i, 0)),
        out_shape=jax.ShapeDtypeStruct((B, L), jnp.int32),
    )(input_ids.astype(jnp.int32), position_ids.astype(jnp.int32),
      types_ids.astype(jnp.int32))

    out = _gather_call()(idx2d.reshape(N), fused)
    return out.reshape(B, L, D)


# speedup vs baseline: 30.4182x; 1.0088x over previous
import functools

import jax
import jax.numpy as jnp
from jax import lax
from jax.experimental import pallas as pl
from jax.experimental.pallas import tpu as pltpu
from jax.experimental.pallas import tpu_sc as plsc

B, L, D = 4096, 200, 128
V_TOK, V_POS, V_TYP = 13, 200, 2
V_FUSED = V_TOK * V_POS * V_TYP
FT = 5248
N = B * L
EPS = 1e-5

NC, NS = 2, 16
NW = NC * NS
PER_W = N // NW
CHUNK = 128
NBUF = 2
NCHUNK = PER_W // CHUNK
NGRP = NCHUNK // NBUF
ROWS_PT = FT // NS


def _prep_body(tok_ref, pos_ref, typ_ref, tok_t, pos_t, typ_t,
               gamma_ref, beta_ref, idx_ref, fused_ref):
    idx_ref[:] = (tok_ref[:] * (V_POS * V_TYP) + pos_ref[:] * V_TYP
                  + typ_ref[:])

    @pl.when(pl.program_id(0) == 0)
    def _():
        jt = lax.broadcasted_iota(jnp.int32, (FT, V_TOK), 0)
        ct = lax.broadcasted_iota(jnp.int32, (FT, V_TOK), 1)
        oh_tok = (jt // (V_POS * V_TYP) == ct).astype(jnp.float32)
        jp = lax.broadcasted_iota(jnp.int32, (FT, V_POS), 0)
        cp = lax.broadcasted_iota(jnp.int32, (FT, V_POS), 1)
        oh_pos = ((jp // V_TYP) % V_POS == cp).astype(jnp.float32)
        e = jnp.dot(oh_tok, tok_t[:], preferred_element_type=jnp.float32,
                    precision=lax.Precision.HIGHEST)
        e = e + jnp.dot(oh_pos, pos_t[:], preferred_element_type=jnp.float32,
                        precision=lax.Precision.HIGHEST)
        jd = lax.broadcasted_iota(jnp.int32, (FT, D), 0)
        e = e + jnp.where(jd % V_TYP == 0, typ_t[0:1, :], typ_t[1:2, :])
        mean = jnp.mean(e, axis=1, keepdims=True)
        cen = e - mean
        var = jnp.mean(cen * cen, axis=1, keepdims=True)
        normed = cen * lax.rsqrt(var + EPS)
        fused_ref[:] = normed * gamma_ref[:] + beta_ref[:]


def _gather_body(idx_hbm, fused_hbm, out_hbm, idx_all, rows, fused_sh,
                 g0, g1, s0, s1):
    gsem = [g0, g1]
    ssem = [s0, s1]
    sid = lax.axis_index("s")
    wid = sid * NC + lax.axis_index("c")
    base = wid * PER_W

    hidx = pltpu.async_copy(idx_hbm.at[pl.ds(base, PER_W)], idx_all, g0)
    pltpu.sync_copy(fused_hbm.at[pl.ds(sid * ROWS_PT, ROWS_PT)],
                    fused_sh.at[pl.ds(sid * ROWS_PT, ROWS_PT)])
    hidx.wait()
    plsc.subcore_barrier()

    def do_group(g, first):
        handles = []
        for b in range(NBUF):
            c = g * NBUF + b
            if not first:
                pltpu.make_async_copy(
                    rows.at[b], out_hbm.at[pl.ds(base + c * CHUNK, CHUNK)],
                    ssem[b]).wait()
            handles.append(pltpu.async_copy(
                fused_sh.at[idx_all.at[pl.ds(c * CHUNK, CHUNK)]],
                rows.at[b], gsem[b]))
        for b in range(NBUF):
            c = g * NBUF + b
            handles[b].wait()
            pltpu.async_copy(rows.at[b],
                             out_hbm.at[pl.ds(base + c * CHUNK, CHUNK)],
                             ssem[b])

    do_group(0, True)

    def body(g, carry):
        do_group(g, False)
        return carry

    lax.fori_loop(1, NGRP, body, 0)
    for b in range(NBUF):
        pltpu.make_async_copy(
            rows.at[b], out_hbm.at[pl.ds(base + b * CHUNK, CHUNK)],
            ssem[b]).wait()


@functools.cache
def _gather_call():
    return pl.kernel(
        _gather_body,
        out_type=jax.ShapeDtypeStruct((N, D), jnp.float32),
        mesh=plsc.VectorSubcoreMesh(core_axis_name="c", subcore_axis_name="s",
                                    num_cores=NC, num_subcores=NS),
        scratch_types=[
            pltpu.VMEM((PER_W,), jnp.int32),
            pltpu.VMEM((NBUF, CHUNK, D), jnp.float32),
            pltpu.VMEM_SHARED((FT, D), jnp.float32),
        ] + [pltpu.SemaphoreType.DMA] * (2 * NBUF),
    )


def kernel(input_ids, position_ids, types_ids, tok_table, pos_table,
           typ_table, ln_gamma, ln_beta):
    blk = 512
    full = lambda i: (0, 0)
    idx2d, fused = pl.pallas_call(
        _prep_body,
        grid=(B // blk,),
        in_specs=[pl.BlockSpec((blk, L), lambda i: (i, 0))] * 3 + [
            pl.BlockSpec((V_TOK, D), full),
            pl.BlockSpec((V_POS, D), full),
            pl.BlockSpec((V_TYP, D), full),
            pl.BlockSpec((1, D), full),
            pl.BlockSpec((1, D), full),
        ],
        out_specs=[
            pl.BlockSpec((blk, L), lambda i: (i, 0)),
            pl.BlockSpec((FT, D), full),
        ],
        out_shape=[
            jax.ShapeDtypeStruct((B, L), jnp.int32),
            jax.ShapeDtypeStruct((FT, D), jnp.float32),
        ],
    )(input_ids.astype(jnp.int32), position_ids.astype(jnp.int32),
      types_ids.astype(jnp.int32), tok_table, pos_table, typ_table,
      ln_gamma.reshape(1, D), ln_beta.reshape(1, D))

    out = _gather_call()(idx2d.reshape(N), fused)
    return out.reshape(B, L, D)


# speedup vs baseline: 44.0553x; 1.4483x over previous
import functools

import jax
import jax.numpy as jnp
from jax import lax
from jax.experimental import pallas as pl
from jax.experimental.pallas import tpu as pltpu
from jax.experimental.pallas import tpu_sc as plsc

B, L, D = 4096, 200, 128
V_TOK, V_POS, V_TYP = 13, 200, 2
V_FUSED = V_TOK * V_POS * V_TYP
FT = 5248
N = B * L
EPS = 1e-5

NC, NS = 2, 16
NW = NC * NS
PER_W = N // NW
CHUNK = 128
NBUF = 3
NCHUNK = PER_W // CHUNK
NGRP = NCHUNK // NBUF
TAIL = NCHUNK - NBUF * NGRP
ROWS_PT = FT // NS


def _prep_body(tok_ref, pos_ref, typ_ref, tok_t, pos_t, typ_t,
               gamma_ref, beta_ref, idx_ref, fused_ref):
    idx_ref[:] = (tok_ref[:] * (V_POS * V_TYP) + pos_ref[:] * V_TYP
                  + typ_ref[:])

    @pl.when(pl.program_id(0) == 0)
    def _():
        pos2 = jnp.broadcast_to(pos_t[:][:, None, :],
                                (V_POS, V_TYP, D)).reshape(V_POS * V_TYP, D)
        e = jnp.concatenate([pos2] * (V_TOK + 1), axis=0)[:FT]
        tok_full = jnp.broadcast_to(tok_t[:][:, None, :],
                                    (V_TOK, V_POS * V_TYP, D)
                                    ).reshape(V_FUSED, D)
        e = e + jnp.concatenate(
            [tok_full, jnp.zeros((FT - V_FUSED, D), jnp.float32)], axis=0)
        jd = lax.broadcasted_iota(jnp.int32, (FT, D), 0)
        e = e + jnp.where(jd % V_TYP == 0, typ_t[0:1, :], typ_t[1:2, :])
        mean = jnp.mean(e, axis=1, keepdims=True)
        cen = e - mean
        var = jnp.mean(cen * cen, axis=1, keepdims=True)
        normed = cen * lax.rsqrt(var + EPS)
        fused_ref[:] = normed * gamma_ref[:] + beta_ref[:]


def _gather_body(idx_hbm, fused_hbm, out_hbm, idx_all, rows, fused_sh,
                 g0, g1, g2, s0, s1, s2):
    gsem = [g0, g1, g2]
    ssem = [s0, s1, s2]
    sid = lax.axis_index("s")
    wid = sid * NC + lax.axis_index("c")
    base = wid * PER_W

    hidx = pltpu.async_copy(idx_hbm.at[pl.ds(base, PER_W)], idx_all, g0)
    pltpu.sync_copy(fused_hbm.at[pl.ds(sid * ROWS_PT, ROWS_PT)],
                    fused_sh.at[pl.ds(sid * ROWS_PT, ROWS_PT)])
    hidx.wait()
    plsc.subcore_barrier()

    def do_group(g, first):
        handles = []
        for b in range(NBUF):
            c = g * NBUF + b
            if not first:
                pltpu.make_async_copy(
                    rows.at[b], out_hbm.at[pl.ds(base + c * CHUNK, CHUNK)],
                    ssem[b]).wait()
            handles.append(pltpu.async_copy(
                fused_sh.at[idx_all.at[pl.ds(c * CHUNK, CHUNK)]],
                rows.at[b], gsem[b]))
        for b in range(NBUF):
            c = g * NBUF + b
            handles[b].wait()
            pltpu.async_copy(rows.at[b],
                             out_hbm.at[pl.ds(base + c * CHUNK, CHUNK)],
                             ssem[b])

    do_group(0, True)

    def body(g, carry):
        do_group(g, False)
        return carry

    lax.fori_loop(1, NGRP, body, 0)
    handles = []
    for b in range(TAIL):
        c = NGRP * NBUF + b
        pltpu.make_async_copy(
            rows.at[b], out_hbm.at[pl.ds(base + c * CHUNK, CHUNK)],
            ssem[b]).wait()
        handles.append(pltpu.async_copy(
            fused_sh.at[idx_all.at[pl.ds(c * CHUNK, CHUNK)]],
            rows.at[b], gsem[b]))
    for b in range(TAIL):
        c = NGRP * NBUF + b
        handles[b].wait()
        pltpu.async_copy(rows.at[b],
                         out_hbm.at[pl.ds(base + c * CHUNK, CHUNK)],
                         ssem[b])
    for b in range(NBUF):
        pltpu.make_async_copy(
            rows.at[b], out_hbm.at[pl.ds(base + b * CHUNK, CHUNK)],
            ssem[b]).wait()


@functools.cache
def _gather_call():
    return pl.kernel(
        _gather_body,
        out_type=jax.ShapeDtypeStruct((N, D), jnp.float32),
        mesh=plsc.VectorSubcoreMesh(core_axis_name="c", subcore_axis_name="s",
                                    num_cores=NC, num_subcores=NS),
        scratch_types=[
            pltpu.VMEM((PER_W,), jnp.int32),
            pltpu.VMEM((NBUF, CHUNK, D), jnp.float32),
            pltpu.VMEM_SHARED((FT, D), jnp.float32),
        ] + [pltpu.SemaphoreType.DMA] * (2 * NBUF),
    )


def kernel(input_ids, position_ids, types_ids, tok_table, pos_table,
           typ_table, ln_gamma, ln_beta):
    blk = 512
    full = lambda i: (0, 0)
    idx2d, fused = pl.pallas_call(
        _prep_body,
        grid=(B // blk,),
        in_specs=[pl.BlockSpec((blk, L), lambda i: (i, 0))] * 3 + [
            pl.BlockSpec((V_TOK, D), full),
            pl.BlockSpec((V_POS, D), full),
            pl.BlockSpec((V_TYP, D), full),
            pl.BlockSpec((1, D), full),
            pl.BlockSpec((1, D), full),
        ],
        out_specs=[
            pl.BlockSpec((blk, L), lambda i: (i, 0)),
            pl.BlockSpec((FT, D), full),
        ],
        out_shape=[
            jax.ShapeDtypeStruct((B, L), jnp.int32),
            jax.ShapeDtypeStruct((FT, D), jnp.float32),
        ],
    )(input_ids.astype(jnp.int32), position_ids.astype(jnp.int32),
      types_ids.astype(jnp.int32), tok_table, pos_table, typ_table,
      ln_gamma.reshape(1, D), ln_beta.reshape(1, D))

    out = _gather_call()(idx2d.reshape(N), fused)
    return out.reshape(B, L, D)
